# Initial kernel scaffold; baseline (speedup 1.0000x reference)
import jax
import jax.numpy as jnp
from jax.experimental import pallas as pl


def kernel(s1, s2, w1, w2, table, W1, b1, W2, b2, W3, b3):
    raise NotImplementedError("write your pallas kernel here")


# speedup vs baseline: 1.6018x; 1.6018x over previous
import functools

import jax
import jax.numpy as jnp
from jax import lax
from jax.experimental import pallas as pl
from jax.experimental.pallas import tpu as pltpu
from jax.experimental.pallas import tpu_sc as plsc

B = 4096
L = 50
D = 64
NC_OUT = 3

_NUM_CORES = 2
_NUM_SUBCORES = 16
_NW = _NUM_CORES * _NUM_SUBCORES
_TASKS = 2 * B
_TPW = _TASKS // _NW
_G = 8
_NG = _TPW // _G
_ROWS_G = _G * L
_PAIRS_G = _G // 2
_LANES = 16
_DCH = D // _LANES


def _make_sc_pool():
    mesh = plsc.VectorSubcoreMesh(core_axis_name="c", subcore_axis_name="s")

    @functools.partial(
        pl.kernel,
        out_type=jax.ShapeDtypeStruct((B, 2 * D), jnp.float32),
        mesh=mesh,
        compiler_params=pltpu.CompilerParams(
            needs_layout_passes=False, use_tc_tiling_on_sc=False),
        scratch_types=[
            pltpu.VMEM((_TPW * L,), jnp.int32),
            pltpu.VMEM((_TPW * L,), jnp.float32),
            pltpu.VMEM((_ROWS_G, D), jnp.float32),
            pltpu.VMEM((_ROWS_G, D), jnp.float32),
            pltpu.VMEM((_PAIRS_G, 2 * D), jnp.float32),
            pltpu.SemaphoreType.DMA,
            pltpu.SemaphoreType.DMA,
        ],
    )
    def pool(s_hbm, w_hbm, table_hbm, out_hbm,
             idx_v, wgt_v, rows_a, rows_b, outb_v, sem_a, sem_b):
        wid = lax.axis_index("s") * _NUM_CORES + lax.axis_index("c")
        ebase = wid * (_TPW * L)

        pltpu.sync_copy(s_hbm.at[pl.ds(ebase, _TPW * L)], idx_v)
        pltpu.sync_copy(w_hbm.at[pl.ds(ebase, _TPW * L)], wgt_v)

        def issue(g, rows, sem):
            base = g * _ROWS_G
            for off, n in ((0, 128), (128, 128), (256, 128), (384, _ROWS_G - 384)):
                pltpu.async_copy(
                    table_hbm.at[idx_v.at[pl.ds(base + off, n)]],
                    rows.at[pl.ds(off, n)],
                    sem,
                )

        def wait(rows, sem):
            pltpu.make_async_copy(table_hbm.at[pl.ds(0, _ROWS_G)], rows, sem).wait()

        def pooled(rows, woff, rbase):
            def body(l, acc):
                wv = plsc.load_gather(
                    wgt_v, [jnp.full((_LANES,), woff + l, jnp.int32)])
                r = rbase + l
                return tuple(acc[c] + wv * rows[r, pl.ds(c * _LANES, _LANES)]
                             for c in range(_DCH))
            z = jnp.zeros((_LANES,), jnp.float32)
            return lax.fori_loop(0, L, body, (z,) * _DCH)

        def compute(g, rows):
            for q in range(_PAIRS_G):
                woff = (g * _G + 2 * q) * L
                e1 = pooled(rows, woff, (2 * q) * L)
                e2 = pooled(rows, woff + L, (2 * q + 1) * L)
                for c in range(_DCH):
                    outb_v[q, pl.ds(c * _LANES, _LANES)] = (
                        jnp.abs(e1[c] - e2[c]) * (1.0 / L))
                    outb_v[q, pl.ds(D + c * _LANES, _LANES)] = (
                        (e1[c] * e2[c]) * (1.0 / (L * L)))
            pair0 = wid * (_TPW // 2) + g * _PAIRS_G
            pltpu.sync_copy(outb_v, out_hbm.at[pl.ds(pair0, _PAIRS_G)])

        issue(0, rows_a, sem_a)
        issue(1, rows_b, sem_b)

        def step(i, carry):
            g0 = 2 * i
            wait(rows_a, sem_a)
            compute(g0, rows_a)

            @pl.when(i < _NG // 2 - 1)
            def _():
                issue(g0 + 2, rows_a, sem_a)

            wait(rows_b, sem_b)
            compute(g0 + 1, rows_b)

            @pl.when(i < _NG // 2 - 1)
            def _():
                issue(g0 + 3, rows_b, sem_b)

            return carry

        lax.fori_loop(0, _NG // 2, step, 0)

    return pool


def _mlp(x, W1, b1, W2, b2, W3, b3):
    def body(x_ref, w1_ref, b1_ref, w2_ref, b2_ref, w3_ref, b3_ref, o_ref):
        f32 = jnp.float32
        w12 = jnp.dot(w1_ref[...], w2_ref[...], preferred_element_type=f32)
        w123 = jnp.dot(w12, w3_ref[...], preferred_element_type=f32)
        b12 = jnp.dot(b1_ref[...], w2_ref[...], preferred_element_type=f32) + b2_ref[...]
        beff = jnp.dot(b12, w3_ref[...], preferred_element_type=f32) + b3_ref[...]
        o_ref[...] = jnp.dot(x_ref[...], w123, preferred_element_type=f32) + beff

    return pl.pallas_call(
        body,
        out_shape=jax.ShapeDtypeStruct((B, NC_OUT), jnp.float32),
    )(x, W1, b1.reshape(1, -1), W2, b2.reshape(1, -1), W3, b3.reshape(1, -1))


def kernel(s1, s2, w1, w2, table, W1, b1, W2, b2, W3, b3):
    s_all = jnp.stack([s1.astype(jnp.int32), s2.astype(jnp.int32)],
                      axis=1).reshape(-1)
    w_all = jnp.stack([w1, w2], axis=1).reshape(-1)
    combine = _make_sc_pool()(s_all, w_all, table)
    return _mlp(combine, W1, b1, W2, b2, W3, b3)
